# Initial kernel scaffold; baseline (speedup 1.0000x reference)
import jax
import jax.numpy as jnp
from jax.experimental import pallas as pl


def kernel(input, tr_mask, tcl_mask, sin_map, cos_map, radii_map, train_mask):
    raise NotImplementedError("write your pallas kernel here")


# speedup vs baseline: 65.5819x; 65.5819x over previous
import functools

import jax
import jax.numpy as jnp
from jax import lax
from jax.experimental import pallas as pl
from jax.experimental.pallas import tpu as pltpu


def _smooth_l1(d):
    return jnp.where(d < 1.0, 0.5 * d * d, d - 0.5)


def _body(in_ref, trm_ref, tclm_ref, sin_ref, cos_ref, rad_ref, trn_ref,
          out_ref, ce_ref, acc_ref, *, hb, w, n_row_chunks, chunk_rows):
    b = pl.program_id(0)
    hc = pl.program_id(1)
    nh = pl.num_programs(1)
    step = b * nh + hc
    last = pl.num_programs(0) * nh - 1

    @pl.when(step == 0)
    def _init():
        for i in range(12):
            acc_ref[i] = 0.0

    fsum = lambda x: jnp.sum(x.astype(jnp.float32))

    l0 = in_ref[0, 0]
    l1 = in_ref[0, 1]
    trm = trm_ref[0] != 0
    trn = trn_ref[0] != 0
    tclm = tclm_ref[0] != 0

    sp = jnp.log(1.0 + jnp.exp(-jnp.abs(l0 - l1)))
    ce_tr = jnp.maximum(l0, l1) + sp - jnp.where(trm, l1, l0)

    pos = trm & trn
    neg = (~trm) & trn
    acc_ref[0] += fsum(pos)
    acc_ref[1] += fsum(neg)
    acc_ref[2] += jnp.sum(jnp.where(pos, ce_tr, 0.0))
    acc_ref[3] += jnp.sum(jnp.where(neg, ce_tr, 0.0))

    l2 = in_ref[0, 2]
    l3 = in_ref[0, 3]
    sp2 = jnp.log(1.0 + jnp.exp(-jnp.abs(l2 - l3)))
    ce_tcl = jnp.maximum(l2, l3) + sp2 - jnp.where(tclm, l3, l2)
    acc_ref[4] += jnp.sum(jnp.where(pos, ce_tcl, 0.0))

    acc_ref[5] += fsum(tclm & trn)
    acc_ref[6] += fsum(tclm)

    s = in_ref[0, 4]
    c = in_ref[0, 5]
    r = in_ref[0, 6]
    scale = lax.rsqrt(s * s + c * c)
    dr = jnp.abs(r / jnp.where(tclm, rad_ref[0], 1.0) - 1.0)
    acc_ref[7] += jnp.sum(jnp.where(tclm, _smooth_l1(dr), 0.0))
    ds = jnp.abs(s * scale - sin_ref[0])
    acc_ref[8] += jnp.sum(jnp.where(tclm, _smooth_l1(ds), 0.0))
    dc = jnp.abs(c * scale - cos_ref[0])
    acc_ref[9] += jnp.sum(jnp.where(tclm, _smooth_l1(dc), 0.0))

    ce_neg_bits = lax.bitcast_convert_type(jnp.where(neg, ce_tr, 0.0), jnp.int32)
    ce_ref[pl.ds(step * hb, hb), :] = ce_neg_bits.reshape(hb, w)

    @pl.when(step == last)
    def _finalize():
        n_pos = acc_ref[0]
        n_negall = acc_ref[1]
        loss_pos = acc_ref[2]
        kf = jnp.where(n_pos > 0.0, jnp.minimum(n_negall, 3.0 * n_pos), 100.0)
        acc_ref[10] = acc_ref[3]

        @pl.when(kf < n_negall)
        def _select():
            k_i = kf.astype(jnp.int32)

            def count_gt(t):
                def cbody(ci, tot):
                    blk = ce_ref[pl.ds(ci * chunk_rows, chunk_rows), :]
                    return tot + jnp.sum((blk > t).astype(jnp.int32))
                return lax.fori_loop(0, n_row_chunks, cbody, jnp.int32(0))

            def bis(_, lohi):
                lo, hi = lohi
                mid = lo + lax.div(hi - lo, jnp.int32(2))
                shrink = count_gt(mid) < k_i
                return jnp.where(shrink, lo, mid), jnp.where(shrink, mid, hi)

            _, t = lax.fori_loop(0, 31, bis,
                                 (jnp.int32(-1), jnp.int32(0x7F800000)))

            def sbody(ci, carry):
                s1, cg = carry
                blk = ce_ref[pl.ds(ci * chunk_rows, chunk_rows), :]
                gt = blk > t
                vals = lax.bitcast_convert_type(blk, jnp.float32)
                return (s1 + jnp.sum(jnp.where(gt, vals, 0.0)),
                        cg + jnp.sum(gt.astype(jnp.int32)))

            s1, cg = lax.fori_loop(0, n_row_chunks, sbody,
                                   (jnp.float32(0.0), jnp.int32(0)))
            tval = lax.bitcast_convert_type(t, jnp.float32)
            acc_ref[10] = s1 + (kf - cg.astype(jnp.float32)) * tval

        out_ref[0] = (loss_pos + acc_ref[10]) / (n_pos + kf)
        out_ref[1] = jnp.where(n_pos > 0.0,
                               acc_ref[4] / jnp.maximum(n_pos, 1.0), 0.0)
        geo_on = acc_ref[5] > 0.0
        n_b = jnp.maximum(acc_ref[6], 1.0)
        out_ref[2] = jnp.where(geo_on, acc_ref[7] / n_b, 0.0)
        out_ref[3] = jnp.where(geo_on, acc_ref[8] / n_b, 0.0)
        out_ref[4] = jnp.where(geo_on, acc_ref[9] / n_b, 0.0)


def kernel(input, tr_mask, tcl_mask, sin_map, cos_map, radii_map, train_mask):
    bs, _, h, w = input.shape
    hb = min(h, 128)
    total_rows = bs * h
    chunk_rows = min(total_rows, 256)
    n_row_chunks = total_rows // chunk_rows

    body = functools.partial(_body, hb=hb, w=w,
                             n_row_chunks=n_row_chunks, chunk_rows=chunk_rows)
    pix_spec = pl.BlockSpec((1, hb, w), lambda b, hc: (b, hc, 0))
    out = pl.pallas_call(
        body,
        grid=(bs, h // hb),
        in_specs=[
            pl.BlockSpec((1, 7, hb, w), lambda b, hc: (b, 0, hc, 0)),
            pix_spec, pix_spec, pix_spec, pix_spec, pix_spec, pix_spec,
        ],
        out_specs=pl.BlockSpec(memory_space=pltpu.SMEM),
        out_shape=jax.ShapeDtypeStruct((8,), jnp.float32),
        scratch_shapes=[
            pltpu.VMEM((total_rows, w), jnp.int32),
            pltpu.SMEM((16,), jnp.float32),
        ],
    )(input, tr_mask, tcl_mask, sin_map, cos_map, radii_map, train_mask)
    return (out[0], out[1], out[2], out[3], out[4])
